# baseline (speedup 1.0000x reference)
import jax
import jax.numpy as jnp
import numpy as np
from jax.experimental import pallas as pl
from jax.experimental.pallas import tpu as pltpu
from jax.sharding import Mesh, PartitionSpec as P

try:
    from jax import shard_map as _shard_map_fn

    def _shard_map(f, mesh, in_specs, out_specs):
        return _shard_map_fn(f, mesh=mesh, in_specs=in_specs,
                             out_specs=out_specs, check_vma=False)
except ImportError:
    from jax.experimental.shard_map import shard_map as _shard_map_fn

    def _shard_map(f, mesh, in_specs, out_specs):
        return _shard_map_fn(f, mesh=mesh, in_specs=in_specs,
                             out_specs=out_specs, check_rep=False)

import functools

BM = 2048
BN = 512
CH = 64


def _x_chunk_copy(x_hbm, stage_ref, sem, row_base, ch, slot):
    return pltpu.make_async_copy(
        x_hbm.at[pl.ds(row_base, ch), :],
        stage_ref.at[slot],
        sem.at[slot],
    )


def _mm_kernel(q, x_hbm, w_ref, b_ref, o_ref, xs_ref, stage_ref, sem):
    i = pl.program_id(0)
    j = pl.program_id(1)
    ni = pl.num_programs(0)
    nj = pl.num_programs(1)
    cur = jax.lax.rem(i, 2)
    nxt = jax.lax.rem(i + 1, 2)
    ct = q * nj

    def _cast(dst, c, slot):
        xs_ref[dst, pl.ds(c * CH, CH), :] = stage_ref[slot].astype(jnp.bfloat16)

    @pl.when((i > 0) & (j == 0))
    def _():
        for k in range(q):
            c = ct - q + k
            slot = jax.lax.rem(c, 2)
            _x_chunk_copy(x_hbm, stage_ref, sem, i * BM + c * CH, CH,
                          slot).wait()
            _cast(cur, c, slot)

    @pl.when((i == 0) & (j == 0))
    def _():
        for c in range(ct):
            _x_chunk_copy(x_hbm, stage_ref, sem, c * CH, CH, c % 2).start()
            if c > 0:
                p = c - 1
                _x_chunk_copy(x_hbm, stage_ref, sem, p * CH, CH, p % 2).wait()
                xs_ref[0, pl.ds(p * CH, CH), :] = (
                    stage_ref[p % 2].astype(jnp.bfloat16))
        c = ct - 1
        _x_chunk_copy(x_hbm, stage_ref, sem, c * CH, CH, c % 2).wait()
        xs_ref[0, pl.ds(c * CH, CH), :] = stage_ref[c % 2].astype(jnp.bfloat16)

    @pl.when(i + 1 < ni)
    def _():
        @pl.when(j > 0)
        def _():
            for k in range(q):
                p = (j - 1) * q + k
                pslot = jax.lax.rem(p, 2)
                _x_chunk_copy(x_hbm, stage_ref, sem, (i + 1) * BM + p * CH,
                              CH, pslot).wait()
                _cast(nxt, p, pslot)

        for k in range(q):
            c = j * q + k
            slot = jax.lax.rem(c, 2)
            _x_chunk_copy(x_hbm, stage_ref, sem, (i + 1) * BM + c * CH,
                          CH, slot).start()

    wb = w_ref[...].astype(jnp.bfloat16)
    acc = jax.lax.dot_general(
        xs_ref[cur], wb,
        dimension_numbers=(((1,), (1,)), ((), ())),
        preferred_element_type=jnp.float32,
    )
    o_ref[...] = acc + b_ref[...]


def _matmul_call(x, weight, bias):
    M, K = x.shape
    N = weight.shape[0]
    nj = N // BN
    assert N % BN == 0 and BM % CH == 0 and (BM // CH) % nj == 0
    q = (BM // CH) // nj
    b2 = bias.reshape(1, N)
    nj1 = nj - 1

    def _serp(i, j):
        return jnp.where(jax.lax.rem(i, 2) == 0, j, nj1 - j)

    return pl.pallas_call(
        functools.partial(_mm_kernel, q),
        grid=(M // BM, N // BN),
        in_specs=[
            pl.BlockSpec(memory_space=pl.ANY),
            pl.BlockSpec((BN, K), lambda i, j: (_serp(i, j), 0)),
            pl.BlockSpec((1, BN), lambda i, j: (0, _serp(i, j))),
        ],
        out_specs=pl.BlockSpec((BM, BN), lambda i, j: (i, _serp(i, j))),
        out_shape=jax.ShapeDtypeStruct((M, N), jnp.float32),
        scratch_shapes=[
            pltpu.VMEM((2, BM, K), jnp.bfloat16),
            pltpu.VMEM((2, CH, K), jnp.float32),
            pltpu.SemaphoreType.DMA((2,)),
        ],
        compiler_params=pltpu.CompilerParams(
            dimension_semantics=("arbitrary", "arbitrary"),
            vmem_limit_bytes=67000320,
        ),
    )(x, weight, b2)


def kernel(x, weight, bias):
    devs = jax.devices()
    n_half = weight.shape[0] // 2
    if (len(devs) < 2 or weight.shape[0] % (2 * BN) != 0
            or (BM // CH) % (n_half // BN) != 0):
        return _matmul_call(x, weight, bias)
    mesh = Mesh(np.array(devs[:2]), ("d",))
    f = _shard_map(
        _matmul_call,
        mesh,
        in_specs=(P(None, None), P("d", None), P("d")),
        out_specs=P(None, "d"),
    )
    return f(x, weight, bias)


# speedup vs baseline: 1.0063x; 1.0063x over previous
import jax
import jax.numpy as jnp
import numpy as np
from jax.experimental import pallas as pl
from jax.experimental.pallas import tpu as pltpu
from jax.sharding import Mesh, PartitionSpec as P

try:
    from jax import shard_map as _shard_map_fn

    def _shard_map(f, mesh, in_specs, out_specs):
        return _shard_map_fn(f, mesh=mesh, in_specs=in_specs,
                             out_specs=out_specs, check_vma=False)
except ImportError:
    from jax.experimental.shard_map import shard_map as _shard_map_fn

    def _shard_map(f, mesh, in_specs, out_specs):
        return _shard_map_fn(f, mesh=mesh, in_specs=in_specs,
                             out_specs=out_specs, check_rep=False)

import functools

BM = 2048
BN = 512
CH = 64


def _x_chunk_copy(x_hbm, stage_ref, sem, row_base, ch, slot):
    return pltpu.make_async_copy(
        x_hbm.at[pl.ds(row_base, ch), :],
        stage_ref.at[slot],
        sem.at[slot],
    )


def _mm_kernel(q, x_hbm, w_ref, b_ref, o_ref, xs_ref, stage_ref, sem):
    i = pl.program_id(0)
    j = pl.program_id(1)
    ni = pl.num_programs(0)
    nj = pl.num_programs(1)
    cur = jax.lax.rem(i, 2)
    nxt = jax.lax.rem(i + 1, 2)
    ct = q * nj

    def _cast(dst, c, slot):
        xs_ref[dst, pl.ds(c * CH, CH), :] = stage_ref[slot].astype(jnp.bfloat16)

    @pl.when((i > 0) & (j == 0))
    def _():
        for k in range(q):
            c = ct - q + k
            slot = jax.lax.rem(c, 2)
            _x_chunk_copy(x_hbm, stage_ref, sem, i * BM + c * CH, CH,
                          slot).wait()
            _cast(cur, c, slot)

    @pl.when((i == 0) & (j == 0))
    def _():
        for c in range(ct):
            _x_chunk_copy(x_hbm, stage_ref, sem, c * CH, CH, c % 2).start()
            if c > 0:
                p = c - 1
                _x_chunk_copy(x_hbm, stage_ref, sem, p * CH, CH, p % 2).wait()
                xs_ref[0, pl.ds(p * CH, CH), :] = (
                    stage_ref[p % 2].astype(jnp.bfloat16))
        c = ct - 1
        _x_chunk_copy(x_hbm, stage_ref, sem, c * CH, CH, c % 2).wait()
        xs_ref[0, pl.ds(c * CH, CH), :] = stage_ref[c % 2].astype(jnp.bfloat16)

    @pl.when(i + 1 < ni)
    def _():
        @pl.when(j > 0)
        def _():
            for k in range(q):
                p = (j - 1) * q + k
                pslot = jax.lax.rem(p, 2)
                _x_chunk_copy(x_hbm, stage_ref, sem, (i + 1) * BM + p * CH,
                              CH, pslot).wait()
                _cast(nxt, p, pslot)

        for k in range(q):
            c = j * q + k
            slot = jax.lax.rem(c, 2)
            _x_chunk_copy(x_hbm, stage_ref, sem, (i + 1) * BM + c * CH,
                          CH, slot).start()

    wb = w_ref[...].astype(jnp.bfloat16)
    acc = jax.lax.dot_general(
        xs_ref[cur], wb,
        dimension_numbers=(((1,), (1,)), ((), ())),
        preferred_element_type=jnp.float32,
    )
    o_ref[...] = acc + b_ref[...]


def _matmul_call(x, weight, bias):
    M, K = x.shape
    N = weight.shape[0]
    nj = N // BN
    assert N % BN == 0 and BM % CH == 0 and (BM // CH) % nj == 0
    q = (BM // CH) // nj
    b2 = bias.reshape(1, N)
    nj1 = nj - 1

    def _serp(i, j):
        return jnp.where(jax.lax.rem(i, 2) == 0, j, nj1 - j)

    return pl.pallas_call(
        functools.partial(_mm_kernel, q),
        grid=(M // BM, N // BN),
        in_specs=[
            pl.BlockSpec(memory_space=pl.ANY),
            pl.BlockSpec((BN, K), lambda i, j: (_serp(i, j), 0)),
            pl.BlockSpec((1, BN), lambda i, j: (0, _serp(i, j))),
        ],
        out_specs=pl.BlockSpec((BM, BN), lambda i, j: (i, _serp(i, j))),
        out_shape=jax.ShapeDtypeStruct((M, N), jnp.float32),
        scratch_shapes=[
            pltpu.VMEM((2, BM, K), jnp.bfloat16),
            pltpu.VMEM((2, CH, K), jnp.float32),
            pltpu.SemaphoreType.DMA((2,)),
        ],
        compiler_params=pltpu.CompilerParams(
            dimension_semantics=("arbitrary", "arbitrary"),
            vmem_limit_bytes=67000320,
        ),
    )(x, weight, b2)


def kernel(x, weight, bias):
    devs = jax.devices()
    if len(devs) < 2 or x.shape[0] % (2 * BM) != 0:
        return _matmul_call(x, weight, bias)
    mesh = Mesh(np.array(devs[:2]), ("d",))
    f = _shard_map(
        _matmul_call,
        mesh,
        in_specs=(P("d", None), P(None, None), P(None)),
        out_specs=P("d", None),
    )
    return f(x, weight, bias)
